# Initial kernel scaffold; baseline (speedup 1.0000x reference)
import jax
import jax.numpy as jnp
from jax.experimental import pallas as pl


def kernel(x0, x1, x2, x3, edge_index0, edge_index1, edge_index2, edge_index3, graph_ids, m0_conv1_W, m0_conv1_b, m0_conv2_W, m0_conv2_b, m0_conv3_W, m0_conv3_b, m0_ec1_tW, m0_ec1_tb, m0_ec1_pW, m0_ec1_pb, m0_ec2_tW, m0_ec2_tb, m0_ec2_pW, m0_ec2_pb, m1_conv1_W, m1_conv1_b, m1_conv2_W, m1_conv2_b, m1_conv3_W, m1_conv3_b, m1_ec1_tW, m1_ec1_tb, m1_ec1_pW, m1_ec1_pb, m1_ec2_tW, m1_ec2_tb, m1_ec2_pW, m1_ec2_pb, m2_conv1_W, m2_conv1_b, m2_conv2_W, m2_conv2_b, m2_conv3_W, m2_conv3_b, m2_ec1_tW, m2_ec1_tb, m2_ec1_pW, m2_ec1_pb, m2_ec2_tW, m2_ec2_tb, m2_ec2_pW, m2_ec2_pb, m3_conv1_W, m3_conv1_b, m3_conv2_W, m3_conv2_b, m3_conv3_W, m3_conv3_b, m3_ec1_tW, m3_ec1_tb, m3_ec1_pW, m3_ec1_pb, m3_ec2_tW, m3_ec2_tb, m3_ec2_pW, m3_ec2_pb, cls_W1, cls_b1, cls_W2, cls_b2, cls_W3, cls_b3):
    raise NotImplementedError("write your pallas kernel here")


# speedup vs baseline: 1.5757x; 1.5757x over previous
import jax
import jax.numpy as jnp
from jax.experimental import pallas as pl

N = 10000
E = 320000
IN_F = 128
H = 64
OUT_F = 16
K = 3
T = 4
B = 128


def _cls_kernel(z_ref, w1_ref, b1_ref, w2_ref, b2_ref, w3_ref, b3_ref, o_ref):
    z = z_ref[...]
    z = jax.nn.relu(jnp.dot(z, w1_ref[...], preferred_element_type=jnp.float32) + b1_ref[...])
    z = jax.nn.relu(jnp.dot(z, w2_ref[...], preferred_element_type=jnp.float32) + b2_ref[...])
    o_ref[...] = jnp.dot(z, w3_ref[...], preferred_element_type=jnp.float32) + b3_ref[...]


def _classifier(z, w1, b1, w2, b2, w3, b3):
    return pl.pallas_call(
        _cls_kernel,
        out_shape=jax.ShapeDtypeStruct((B, OUT_F), jnp.float32),
    )(z, w1, b1, w2, b2, w3, b3)


def _seg_sum(vals, dst):
    return jax.ops.segment_sum(vals, dst, num_segments=N)


def _tower(x, src, dst, p):
    deg = _seg_sum(jnp.ones((E,), jnp.float32), dst)
    norm = jnp.power(jnp.clip(deg, 1.0), -0.5)[:, None]
    normsq = norm * norm

    def prop(v):
        return _seg_sum(v[src], dst)

    def cheb(h, W, b):
        W0, W1, W2 = W[:h.shape[1]], W[h.shape[1]:2 * h.shape[1]], W[2 * h.shape[1]:]
        C = h @ (W0 - W2) + b
        A = norm * (h @ W1)
        B2 = 2.0 * (norm * (h @ W2))
        U = prop(B2)
        m = normsq * U - A
        V = norm * prop(m)
        return jax.nn.relu(C + V)

    def edge_conv(h, tW, tb, pW, pb):
        G = h @ tW
        Q = h @ (tW + pW) + (tb + pb)
        M = jax.ops.segment_min(G[src], dst, num_segments=N)
        return jax.nn.relu(jnp.where(deg[:, None] > 0, Q - M, 0.0))

    h = cheb(x, p("conv1_W"), p("conv1_b"))
    h = edge_conv(h, p("ec1_tW"), p("ec1_tb"), p("ec1_pW"), p("ec1_pb"))
    h = cheb(h, p("conv2_W"), p("conv2_b"))
    h = edge_conv(h, p("ec2_tW"), p("ec2_tb"), p("ec2_pW"), p("ec2_pb"))
    h = cheb(h, p("conv3_W"), p("conv3_b"))
    return h


def kernel(x0, x1, x2, x3, edge_index0, edge_index1, edge_index2, edge_index3, graph_ids, m0_conv1_W, m0_conv1_b, m0_conv2_W, m0_conv2_b, m0_conv3_W, m0_conv3_b, m0_ec1_tW, m0_ec1_tb, m0_ec1_pW, m0_ec1_pb, m0_ec2_tW, m0_ec2_tb, m0_ec2_pW, m0_ec2_pb, m1_conv1_W, m1_conv1_b, m1_conv2_W, m1_conv2_b, m1_conv3_W, m1_conv3_b, m1_ec1_tW, m1_ec1_tb, m1_ec1_pW, m1_ec1_pb, m1_ec2_tW, m1_ec2_tb, m1_ec2_pW, m1_ec2_pb, m2_conv1_W, m2_conv1_b, m2_conv2_W, m2_conv2_b, m2_conv3_W, m2_conv3_b, m2_ec1_tW, m2_ec1_tb, m2_ec1_pW, m2_ec1_pb, m2_ec2_tW, m2_ec2_tb, m2_ec2_pW, m2_ec2_pb, m3_conv1_W, m3_conv1_b, m3_conv2_W, m3_conv2_b, m3_conv3_W, m3_conv3_b, m3_ec1_tW, m3_ec1_tb, m3_ec1_pW, m3_ec1_pb, m3_ec2_tW, m3_ec2_tb, m3_ec2_pW, m3_ec2_pb, cls_W1, cls_b1, cls_W2, cls_b2, cls_W3, cls_b3):
    fl = dict(locals())
    xs = [x0, x1, x2, x3]
    eis = [edge_index0, edge_index1, edge_index2, edge_index3]

    gid = graph_ids
    cnt = jnp.clip(jax.ops.segment_sum(jnp.ones((N,), jnp.float32), gid, num_segments=B), 1.0)[:, None]
    onehot = (gid[None, :] == jnp.arange(B, dtype=jnp.int32)[:, None]).astype(jnp.float32)

    reps = []
    for t in range(T):
        p = lambda nm, t=t: fl[f"m{t}_{nm}"]
        h = _tower(xs[t], eis[t][0], eis[t][1], p)
        reps.append((onehot @ h) / cnt)
    z = jnp.concatenate(reps, axis=1)
    return _classifier(z, cls_W1, cls_b1, cls_W2, cls_b2, cls_W3, cls_b3)


# speedup vs baseline: 3.8889x; 2.4681x over previous
import jax
import jax.numpy as jnp
from jax import lax
from jax.experimental import pallas as pl
from jax.experimental.pallas import tpu as pltpu
from jax.experimental.pallas import tpu_sc as plsc

N = 10000
E = 320000
IN_F = 128
H = 64
OUT_F = 16
K = 3
T = 4
B = 128

NSUB = 16
EPS = E // NSUB
CH = 128
NFULL = EPS // CH
REM = EPS - NFULL * CH
NP = 10240
NPS = NP // NSUB

RNG = NP // 8
ACC_R = 1296
SENT_DL = 1288
NSEG = 8
SEGE = E // NSEG
SCH = 400
CAP = 5632
CAPB = CAP + 16
GB = 128
BIG = 3.0e38

_f32 = jnp.float32
_i32 = jnp.int32


def _mesh():
    return plsc.VectorSubcoreMesh(core_axis_name="c", subcore_axis_name="s",
                                  num_cores=2, num_subcores=NSUB)


def _iota16():
    return lax.iota(_i32, 16)


def _deg_body(dst0, dst1, dst2, dst3, zeros16, ones_h,
              deg0, deg1, deg2, deg3,
              acc16, idxb, idxr, onesb):
    c = lax.axis_index("c")
    s = lax.axis_index("s")
    dsts = (dst0, dst1, dst2, dst3)
    outs = (deg0, deg1, deg2, deg3)
    pltpu.sync_copy(ones_h, onesb)
    for t in range(T):
        @pl.when(c == t // 2)
        def _(t=t):
            pltpu.sync_copy(zeros16.at[pl.ds(s * NPS, NPS)],
                            acc16.at[pl.ds(s * NPS, NPS)])
            plsc.subcore_barrier()

            def chunk(k, _):
                pltpu.sync_copy(dsts[t].at[pl.ds(s * EPS + k * CH, CH)], idxb)
                pltpu.sync_copy(onesb, acc16.at[idxb], add=True)
                return 0

            lax.fori_loop(0, NFULL, chunk, 0)
            pltpu.sync_copy(dsts[t].at[pl.ds(s * EPS + NFULL * CH, REM)], idxr)
            pltpu.sync_copy(onesb.at[pl.ds(0, REM)], acc16.at[idxr], add=True)
            plsc.subcore_barrier()
            pltpu.sync_copy(acc16.at[pl.ds(s * NPS, NPS)],
                            outs[t].at[pl.ds(s * NPS, NPS)])
            plsc.subcore_barrier()


def _sc_deg(dsts, zeros16, ones_h):
    f = pl.kernel(
        _deg_body,
        out_type=[jax.ShapeDtypeStruct((NP, 16), _f32) for _ in range(T)],
        mesh=_mesh(),
        compiler_params=pltpu.CompilerParams(use_tc_tiling_on_sc=False, needs_layout_passes=False),
        scratch_types=[
            pltpu.VMEM_SHARED((NP, 16), _f32),
            pltpu.VMEM((CH,), _i32),
            pltpu.VMEM((REM,), _i32),
            pltpu.VMEM((CH, 16), _f32),
        ],
    )
    return f(dsts[0], dsts[1], dsts[2], dsts[3], zeros16, ones_h)


def _cheb_scatter_pass(vsrc, src_h, dst_h, accS, s,
                       srcb0, dstb0, srcb1, dstb1, rows0, rows1,
                       srcr, dstr, rowsr, sem0, sem1):
    def pair(i, _):
        o0 = s * EPS + (2 * i) * CH
        pltpu.sync_copy(src_h.at[pl.ds(o0, CH)], srcb0)
        pltpu.sync_copy(dst_h.at[pl.ds(o0, CH)], dstb0)
        d0 = pltpu.async_copy(vsrc.at[srcb0], rows0, sem0)
        o1 = o0 + CH
        pltpu.sync_copy(src_h.at[pl.ds(o1, CH)], srcb1)
        pltpu.sync_copy(dst_h.at[pl.ds(o1, CH)], dstb1)
        d1 = pltpu.async_copy(vsrc.at[srcb1], rows1, sem1)
        d0.wait()
        pltpu.sync_copy(rows0, accS.at[dstb0], add=True)
        d1.wait()
        pltpu.sync_copy(rows1, accS.at[dstb1], add=True)
        return 0

    lax.fori_loop(0, NFULL // 2, pair, 0)
    orr = s * EPS + NFULL * CH
    pltpu.sync_copy(src_h.at[pl.ds(orr, REM)], srcr)
    pltpu.sync_copy(dst_h.at[pl.ds(orr, REM)], dstr)
    pltpu.async_copy(vsrc.at[srcr], rowsr, sem0).wait()
    pltpu.sync_copy(rowsr, accS.at[dstr], add=True)


def _cheb_body(*refs):
    (b0, b1, b2, b3, a0, a1, a2, a3, n0, n1, n2, n3,
     s0, s1, s2, s3, d0, d1, d2, d3, zeros64,
     m0, m1, m2, m3, o0, o1, o2, o3,
     accS, srcb0, dstb0, srcb1, dstb1, rows0, rows1,
     srcr, dstr, rowsr, cb, nb, ab, sem0, sem1) = refs
    c = lax.axis_index("c")
    s = lax.axis_index("s")
    Bs = (b0, b1, b2, b3)
    As = (a0, a1, a2, a3)
    Ns = (n0, n1, n2, n3)
    Ss = (s0, s1, s2, s3)
    Ds = (d0, d1, d2, d3)
    Ms = (m0, m1, m2, m3)
    Os = (o0, o1, o2, o3)
    for t in range(T):
        @pl.when(c == t // 2)
        def _(t=t):
            sl = pl.ds(s * NPS, NPS)
            pltpu.sync_copy(zeros64.at[sl], accS.at[sl])
            plsc.subcore_barrier()
            _cheb_scatter_pass(Bs[t], Ss[t], Ds[t], accS, s,
                               srcb0, dstb0, srcb1, dstb1, rows0, rows1,
                               srcr, dstr, rowsr, sem0, sem1)
            plsc.subcore_barrier()

            def mchunk(j, _):
                off = s * NPS + j * 128
                osl = pl.ds(off, 128)
                pltpu.sync_copy(accS.at[osl], cb)
                pltpu.sync_copy(Ns[t].at[osl], nb)
                pltpu.sync_copy(As[t].at[osl], ab)

                def mrow(r, _2):
                    for f in range(4):
                        fs = pl.ds(16 * f, 16)
                        cb[r, fs] = nb[r, fs] * cb[r, fs] - ab[r, fs]
                    return 0

                lax.fori_loop(0, 128, mrow, 0)
                pltpu.sync_copy(cb, Ms[t].at[osl])
                return 0

            lax.fori_loop(0, 5, mchunk, 0)
            pltpu.sync_copy(zeros64.at[sl], accS.at[sl])
            plsc.subcore_barrier()
            _cheb_scatter_pass(Ms[t], Ss[t], Ds[t], accS, s,
                               srcb0, dstb0, srcb1, dstb1, rows0, rows1,
                               srcr, dstr, rowsr, sem0, sem1)
            plsc.subcore_barrier()
            pltpu.sync_copy(accS.at[sl], Os[t].at[sl])


def _sc_cheb(Bp, A, NSq, srcs, dsts, zeros64):
    f = pl.kernel(
        _cheb_body,
        out_type=[jax.ShapeDtypeStruct((NP, H), _f32) for _ in range(2 * T)],
        mesh=_mesh(),
        compiler_params=pltpu.CompilerParams(use_tc_tiling_on_sc=False, needs_layout_passes=False),
        scratch_types=[
            pltpu.VMEM_SHARED((NP, H), _f32),
            pltpu.VMEM((CH,), _i32), pltpu.VMEM((CH,), _i32),
            pltpu.VMEM((CH,), _i32), pltpu.VMEM((CH,), _i32),
            pltpu.VMEM((CH, H), _f32), pltpu.VMEM((CH, H), _f32),
            pltpu.VMEM((REM,), _i32), pltpu.VMEM((REM,), _i32),
            pltpu.VMEM((REM, H), _f32),
            pltpu.VMEM((128, H), _f32), pltpu.VMEM((128, H), _f32),
            pltpu.VMEM((128, H), _f32),
            pltpu.SemaphoreType.DMA, pltpu.SemaphoreType.DMA,
        ],
    )
    outs = f(Bp[0], Bp[1], Bp[2], Bp[3], A[0], A[1], A[2], A[3],
             NSq[0], NSq[1], NSq[2], NSq[3],
             srcs[0], srcs[1], srcs[2], srcs[3],
             dsts[0], dsts[1], dsts[2], dsts[3], zeros64)
    return outs[T:]


def _emin_tower(G_h, src_h, dst_h, M_h, w, t,
                acc, sbuf, dbuf, fsrc, fdl, rows0, rows1, sem0, sem1):
    r = w - 8 * t
    lo = r * RNG

    def arow(i, _):
        for f in range(4):
            acc[i, pl.ds(16 * f, 16)] = jnp.full((16,), BIG, _f32)
        return 0

    lax.fori_loop(0, ACC_R, arow, 0)

    def seg(ss, _):
        def pre(i, _2):
            fdl[pl.ds(16 * i, 16)] = jnp.full((16,), SENT_DL, _i32)
            fsrc[pl.ds(16 * i, 16)] = jnp.full((16,), N, _i32)
            return 0

        lax.fori_loop(0, CAPB // 16, pre, 0)

        def schunk(k, cur):
            off = ss * SEGE + k * SCH
            pltpu.sync_copy(src_h.at[pl.ds(off, SCH)], sbuf)
            pltpu.sync_copy(dst_h.at[pl.ds(off, SCH)], dbuf)

            def inner(j, cur2):
                dv = dbuf[pl.ds(16 * j, 16)]
                sv = sbuf[pl.ds(16 * j, 16)]
                msk = (dv >= lo) & (dv < lo + RNG)
                mi = msk.astype(_i32)
                pos = cur2 + plsc.cumsum(mi) - 1
                plsc.store_scatter(fdl, [pos], dv - lo, mask=msk)
                plsc.store_scatter(fsrc, [pos], sv, mask=msk)
                return cur2 + jnp.sum(mi)

            return lax.fori_loop(0, SCH // 16, inner, cur)

        lax.fori_loop(0, SEGE // SCH, schunk, 0)

        def rmw(g, rows, lbase):
            base = g * 16

            def edge(e):
                dl = lax.gather(
                    fdl[pl.ds(lbase + base, 16)],
                    jnp.full((16, 1), e, _i32),
                    lax.GatherDimensionNumbers(offset_dims=(),
                                               collapsed_slice_dims=(0,),
                                               start_index_map=(0,)),
                    (1,),
                    mode=lax.GatherScatterMode.PROMISE_IN_BOUNDS)
                for f in range(4):
                    col = _iota16() + 16 * f
                    av = plsc.load_gather(acc, [dl, col])
                    mv = rows[base + e, pl.ds(16 * f, 16)]
                    plsc.store_scatter(acc, [dl, col], jnp.minimum(av, mv))

            for e in range(16):
                edge(e)

        def bpair(i, _2):
            b0 = (2 * i) * GB
            d0 = pltpu.async_copy(G_h.at[fsrc.at[pl.ds(b0, GB)]], rows0, sem0)
            b1 = b0 + GB
            d1 = pltpu.async_copy(G_h.at[fsrc.at[pl.ds(b1, GB)]], rows1, sem1)
            d0.wait()

            def g0(g, _3):
                rmw(g, rows0, b0)
                return 0

            lax.fori_loop(0, GB // 16, g0, 0)
            d1.wait()

            def g1(g, _3):
                rmw(g, rows1, b1)
                return 0

            lax.fori_loop(0, GB // 16, g1, 0)
            return 0

        lax.fori_loop(0, CAP // GB // 2, bpair, 0)
        return 0

    lax.fori_loop(0, NSEG, seg, 0)
    pltpu.sync_copy(acc.at[pl.ds(0, RNG)], M_h.at[pl.ds(lo, RNG)])


def _emin_body(*refs):
    (g0, g1, g2, g3, s0, s1, s2, s3, d0, d1, d2, d3,
     m0, m1, m2, m3,
     acc, sbuf, dbuf, fsrc, fdl, rows0, rows1, sem0, sem1) = refs
    c = lax.axis_index("c")
    s = lax.axis_index("s")
    w = c * NSUB + s
    Gs = (g0, g1, g2, g3)
    Ss = (s0, s1, s2, s3)
    Ds = (d0, d1, d2, d3)
    Ms = (m0, m1, m2, m3)
    for t in range(T):
        @pl.when(w // 8 == t)
        def _(t=t):
            _emin_tower(Gs[t], Ss[t], Ds[t], Ms[t], w, t,
                        acc, sbuf, dbuf, fsrc, fdl, rows0, rows1, sem0, sem1)


def _sc_emin(Gp, srcs, dsts):
    f = pl.kernel(
        _emin_body,
        out_type=[jax.ShapeDtypeStruct((NP, H), _f32) for _ in range(T)],
        mesh=_mesh(),
        compiler_params=pltpu.CompilerParams(use_tc_tiling_on_sc=False, needs_layout_passes=False),
        scratch_types=[
            pltpu.VMEM((ACC_R, H), _f32),
            pltpu.VMEM((SCH,), _i32), pltpu.VMEM((SCH,), _i32),
            pltpu.VMEM((CAPB,), _i32), pltpu.VMEM((CAPB,), _i32),
            pltpu.VMEM((GB, H), _f32), pltpu.VMEM((GB, H), _f32),
            pltpu.SemaphoreType.DMA, pltpu.SemaphoreType.DMA,
        ],
    )
    return f(Gp[0], Gp[1], Gp[2], Gp[3],
             srcs[0], srcs[1], srcs[2], srcs[3],
             dsts[0], dsts[1], dsts[2], dsts[3])


def _cls_kernel(z_ref, w1_ref, b1_ref, w2_ref, b2_ref, w3_ref, b3_ref, o_ref):
    z = z_ref[...]
    z = jax.nn.relu(jnp.dot(z, w1_ref[...], preferred_element_type=_f32) + b1_ref[...])
    z = jax.nn.relu(jnp.dot(z, w2_ref[...], preferred_element_type=_f32) + b2_ref[...])
    o_ref[...] = jnp.dot(z, w3_ref[...], preferred_element_type=_f32) + b3_ref[...]


def _classifier(z, w1, b1, w2, b2, w3, b3):
    return pl.pallas_call(
        _cls_kernel,
        out_shape=jax.ShapeDtypeStruct((B, OUT_F), _f32),
    )(z, w1, b1, w2, b2, w3, b3)


def kernel(x0, x1, x2, x3, edge_index0, edge_index1, edge_index2, edge_index3, graph_ids, m0_conv1_W, m0_conv1_b, m0_conv2_W, m0_conv2_b, m0_conv3_W, m0_conv3_b, m0_ec1_tW, m0_ec1_tb, m0_ec1_pW, m0_ec1_pb, m0_ec2_tW, m0_ec2_tb, m0_ec2_pW, m0_ec2_pb, m1_conv1_W, m1_conv1_b, m1_conv2_W, m1_conv2_b, m1_conv3_W, m1_conv3_b, m1_ec1_tW, m1_ec1_tb, m1_ec1_pW, m1_ec1_pb, m1_ec2_tW, m1_ec2_tb, m1_ec2_pW, m1_ec2_pb, m2_conv1_W, m2_conv1_b, m2_conv2_W, m2_conv2_b, m2_conv3_W, m2_conv3_b, m2_ec1_tW, m2_ec1_tb, m2_ec1_pW, m2_ec1_pb, m2_ec2_tW, m2_ec2_tb, m2_ec2_pW, m2_ec2_pb, m3_conv1_W, m3_conv1_b, m3_conv2_W, m3_conv2_b, m3_conv3_W, m3_conv3_b, m3_ec1_tW, m3_ec1_tb, m3_ec1_pW, m3_ec1_pb, m3_ec2_tW, m3_ec2_tb, m3_ec2_pW, m3_ec2_pb, cls_W1, cls_b1, cls_W2, cls_b2, cls_W3, cls_b3):
    fl = dict(locals())
    xs = [x0, x1, x2, x3]
    eis = [edge_index0, edge_index1, edge_index2, edge_index3]
    srcs = [ei[0] for ei in eis]
    dsts = [ei[1] for ei in eis]

    zeros16 = jnp.zeros((NP, 16), _f32)
    zeros64 = jnp.zeros((NP, H), _f32)
    ones_h = jnp.ones((CH, 16), _f32)
    padrows = jnp.zeros((NP - N, H), _f32)
    pad = lambda a: jnp.concatenate([a, padrows], axis=0)

    deg16 = _sc_deg(dsts, zeros16, ones_h)
    degs = [d[:N, 0] for d in deg16]
    norms = [jnp.power(jnp.clip(d, 1.0), -0.5)[:, None] for d in degs]
    nsq64 = [pad(jnp.broadcast_to(nm * nm, (N, H))) for nm in norms]

    def cheb_stage(hs, Wn, bn):
        Bp, A, C = [], [], []
        for t in range(T):
            W = fl[f"m{t}_{Wn}"]
            b = fl[f"m{t}_{bn}"]
            F = hs[t].shape[1]
            W0, W1, W2 = W[:F], W[F:2 * F], W[2 * F:]
            C.append(hs[t] @ (W0 - W2) + b)
            A.append(pad(norms[t] * (hs[t] @ W1)))
            Bp.append(pad(2.0 * (norms[t] * (hs[t] @ W2))))
        S = _sc_cheb(Bp, A, nsq64, srcs, dsts, zeros64)
        return [jax.nn.relu(C[t] + norms[t] * S[t][:N]) for t in range(T)]

    def emin_stage(hs, tWn, tbn, pWn, pbn):
        Gp, Q = [], []
        pad = jnp.full((16, H), BIG, _f32)
        for t in range(T):
            tW = fl[f"m{t}_{tWn}"]
            pW = fl[f"m{t}_{pWn}"]
            cst = fl[f"m{t}_{tbn}"] + fl[f"m{t}_{pbn}"]
            Gp.append(jnp.concatenate([hs[t] @ tW, pad], axis=0))
            Q.append(hs[t] @ (tW + pW) + cst)
        M = _sc_emin(Gp, srcs, dsts)
        return [jax.nn.relu(jnp.where(degs[t][:, None] > 0, Q[t] - M[t][:N], 0.0))
                for t in range(T)]

    h = cheb_stage(xs, "conv1_W", "conv1_b")
    h = emin_stage(h, "ec1_tW", "ec1_tb", "ec1_pW", "ec1_pb")
    h = cheb_stage(h, "conv2_W", "conv2_b")
    h = emin_stage(h, "ec2_tW", "ec2_tb", "ec2_pW", "ec2_pb")
    h = cheb_stage(h, "conv3_W", "conv3_b")

    gid = graph_ids
    cnt = jnp.clip(jax.ops.segment_sum(jnp.ones((N,), _f32), gid, num_segments=B), 1.0)[:, None]
    onehot = (gid[None, :] == jnp.arange(B, dtype=_i32)[:, None]).astype(_f32)
    reps = [(onehot @ h[t]) / cnt for t in range(T)]
    z = jnp.concatenate(reps, axis=1)
    return _classifier(z, cls_W1, cls_b1, cls_W2, cls_b2, cls_W3, cls_b3)


# speedup vs baseline: 3.9047x; 1.0041x over previous
import jax
import jax.numpy as jnp
from jax import lax
from jax.experimental import pallas as pl
from jax.experimental.pallas import tpu as pltpu
from jax.experimental.pallas import tpu_sc as plsc

N = 10000
E = 320000
IN_F = 128
H = 64
OUT_F = 16
K = 3
T = 4
B = 128

NSUB = 16
EPS = E // NSUB
CH = 128
NFULL = EPS // CH
REM = EPS - NFULL * CH
CEPS = E // 8
CNF = CEPS // CH
CREM = CEPS - CNF * CH
CNPS = 1280
NP = 10240
NPS = NP // NSUB

RNG = NP // 8
ACC_R = 1296
SENT_DL = 1288
NSEG = 8
SEGE = E // NSEG
SCH = 400
CAP = 5632
CAPB = CAP + 16
GB = 128
BIG = 3.0e38

_f32 = jnp.float32
_i32 = jnp.int32


def _mesh():
    return plsc.VectorSubcoreMesh(core_axis_name="c", subcore_axis_name="s",
                                  num_cores=2, num_subcores=NSUB)


def _iota16():
    return lax.iota(_i32, 16)


def _deg_body(dst0, dst1, dst2, dst3, zeros16, ones_h,
              deg0, deg1, deg2, deg3,
              acc16, idxb, idxr, onesb):
    c = lax.axis_index("c")
    s = lax.axis_index("s")
    dsts = (dst0, dst1, dst2, dst3)
    outs = (deg0, deg1, deg2, deg3)
    pltpu.sync_copy(ones_h, onesb)
    for t in range(T):
        @pl.when(c == t // 2)
        def _(t=t):
            pltpu.sync_copy(zeros16.at[pl.ds(s * NPS, NPS)],
                            acc16.at[pl.ds(s * NPS, NPS)])
            plsc.subcore_barrier()

            def chunk(k, _):
                pltpu.sync_copy(dsts[t].at[pl.ds(s * EPS + k * CH, CH)], idxb)
                pltpu.sync_copy(onesb, acc16.at[idxb], add=True)
                return 0

            lax.fori_loop(0, NFULL, chunk, 0)
            pltpu.sync_copy(dsts[t].at[pl.ds(s * EPS + NFULL * CH, REM)], idxr)
            pltpu.sync_copy(onesb.at[pl.ds(0, REM)], acc16.at[idxr], add=True)
            plsc.subcore_barrier()
            pltpu.sync_copy(acc16.at[pl.ds(s * NPS, NPS)],
                            outs[t].at[pl.ds(s * NPS, NPS)])
            plsc.subcore_barrier()


def _sc_deg(dsts, zeros16, ones_h):
    f = pl.kernel(
        _deg_body,
        out_type=[jax.ShapeDtypeStruct((NP, 16), _f32) for _ in range(T)],
        mesh=_mesh(),
        compiler_params=pltpu.CompilerParams(use_tc_tiling_on_sc=False, needs_layout_passes=False),
        scratch_types=[
            pltpu.VMEM_SHARED((NP, 16), _f32),
            pltpu.VMEM((CH,), _i32),
            pltpu.VMEM((REM,), _i32),
            pltpu.VMEM((CH, 16), _f32),
        ],
    )
    return f(dsts[0], dsts[1], dsts[2], dsts[3], zeros16, ones_h)


def _cheb_scatter_pass(vsrc, src_h, dst_h, accS, sub,
                       srcb0, dstb0, srcb1, dstb1, rows0, rows1,
                       srcr, dstr, rowsr, sem0, sem1):
    def pair(i, _):
        o0 = sub * CEPS + (2 * i) * CH
        pltpu.sync_copy(src_h.at[pl.ds(o0, CH)], srcb0)
        pltpu.sync_copy(dst_h.at[pl.ds(o0, CH)], dstb0)
        d0 = pltpu.async_copy(vsrc.at[srcb0], rows0, sem0)
        o1 = o0 + CH
        pltpu.sync_copy(src_h.at[pl.ds(o1, CH)], srcb1)
        pltpu.sync_copy(dst_h.at[pl.ds(o1, CH)], dstb1)
        d1 = pltpu.async_copy(vsrc.at[srcb1], rows1, sem1)
        d0.wait()
        pltpu.sync_copy(rows0, accS.at[dstb0], add=True)
        d1.wait()
        pltpu.sync_copy(rows1, accS.at[dstb1], add=True)
        return 0

    lax.fori_loop(0, CNF // 2, pair, 0)
    orr = sub * CEPS + CNF * CH
    pltpu.sync_copy(src_h.at[pl.ds(orr, CREM)], srcr)
    pltpu.sync_copy(dst_h.at[pl.ds(orr, CREM)], dstr)
    pltpu.async_copy(vsrc.at[srcr], rowsr, sem0).wait()
    pltpu.sync_copy(rowsr, accS.at[dstr], add=True)


def _cheb_body(*refs):
    (b0, b1, b2, b3, a0, a1, a2, a3, n0, n1, n2, n3,
     s0, s1, s2, s3, d0, d1, d2, d3, zeros64,
     m0, m1, m2, m3, o0, o1, o2, o3,
     accA, accB, srcb0, dstb0, srcb1, dstb1, rows0, rows1,
     srcr, dstr, rowsr, cb, nb, ab, sem0, sem1) = refs
    c = lax.axis_index("c")
    s = lax.axis_index("s")
    gi = s // 8
    sub = s - 8 * gi
    Bs = (b0, b1, b2, b3)
    As = (a0, a1, a2, a3)
    Ns = (n0, n1, n2, n3)
    Ss = (s0, s1, s2, s3)
    Ds = (d0, d1, d2, d3)
    Ms = (m0, m1, m2, m3)
    Os = (o0, o1, o2, o3)
    for t in range(T):
        @pl.when((c == t // 2) & (gi == t % 2))
        def _(t=t):
            accS = (accA, accB)[t % 2]
            sl = pl.ds(sub * CNPS, CNPS)
            pltpu.sync_copy(zeros64.at[sl], accS.at[sl])
            plsc.subcore_barrier()
            _cheb_scatter_pass(Bs[t], Ss[t], Ds[t], accS, sub,
                               srcb0, dstb0, srcb1, dstb1, rows0, rows1,
                               srcr, dstr, rowsr, sem0, sem1)
            plsc.subcore_barrier()

            def mchunk(j, _):
                off = sub * CNPS + j * 128
                osl = pl.ds(off, 128)
                pltpu.sync_copy(accS.at[osl], cb)
                pltpu.sync_copy(Ns[t].at[osl], nb)
                pltpu.sync_copy(As[t].at[osl], ab)

                def mrow(r, _2):
                    for f in range(4):
                        fs = pl.ds(16 * f, 16)
                        cb[r, fs] = nb[r, fs] * cb[r, fs] - ab[r, fs]
                    return 0

                lax.fori_loop(0, 128, mrow, 0)
                pltpu.sync_copy(cb, Ms[t].at[osl])
                return 0

            lax.fori_loop(0, 10, mchunk, 0)
            pltpu.sync_copy(zeros64.at[sl], accS.at[sl])
            plsc.subcore_barrier()
            _cheb_scatter_pass(Ms[t], Ss[t], Ds[t], accS, sub,
                               srcb0, dstb0, srcb1, dstb1, rows0, rows1,
                               srcr, dstr, rowsr, sem0, sem1)
            plsc.subcore_barrier()
            pltpu.sync_copy(accS.at[sl], Os[t].at[sl])


def _sc_cheb(Bp, A, NSq, srcs, dsts, zeros64):
    f = pl.kernel(
        _cheb_body,
        out_type=[jax.ShapeDtypeStruct((NP, H), _f32) for _ in range(2 * T)],
        mesh=_mesh(),
        compiler_params=pltpu.CompilerParams(use_tc_tiling_on_sc=False, needs_layout_passes=False),
        scratch_types=[
            pltpu.VMEM_SHARED((NP, H), _f32),
            pltpu.VMEM_SHARED((NP, H), _f32),
            pltpu.VMEM((CH,), _i32), pltpu.VMEM((CH,), _i32),
            pltpu.VMEM((CH,), _i32), pltpu.VMEM((CH,), _i32),
            pltpu.VMEM((CH, H), _f32), pltpu.VMEM((CH, H), _f32),
            pltpu.VMEM((CREM,), _i32), pltpu.VMEM((CREM,), _i32),
            pltpu.VMEM((CREM, H), _f32),
            pltpu.VMEM((128, H), _f32), pltpu.VMEM((128, H), _f32),
            pltpu.VMEM((128, H), _f32),
            pltpu.SemaphoreType.DMA, pltpu.SemaphoreType.DMA,
        ],
    )
    outs = f(Bp[0], Bp[1], Bp[2], Bp[3], A[0], A[1], A[2], A[3],
             NSq[0], NSq[1], NSq[2], NSq[3],
             srcs[0], srcs[1], srcs[2], srcs[3],
             dsts[0], dsts[1], dsts[2], dsts[3], zeros64)
    return outs[T:]


def _emin_tower(G_h, src_h, dst_h, M_h, w, t,
                acc, sbuf, dbuf, fsrc, fdl, rows0, rows1, sem0, sem1):
    r = w - 8 * t
    lo = r * RNG

    def arow(i, _):
        for f in range(4):
            acc[i, pl.ds(16 * f, 16)] = jnp.full((16,), BIG, _f32)
        return 0

    lax.fori_loop(0, ACC_R, arow, 0)

    def seg(ss, _):
        def pre(i, _2):
            fdl[pl.ds(16 * i, 16)] = jnp.full((16,), SENT_DL, _i32)
            fsrc[pl.ds(16 * i, 16)] = jnp.full((16,), N, _i32)
            return 0

        lax.fori_loop(0, CAPB // 16, pre, 0)

        def schunk(k, cur):
            off = ss * SEGE + k * SCH
            pltpu.sync_copy(src_h.at[pl.ds(off, SCH)], sbuf)
            pltpu.sync_copy(dst_h.at[pl.ds(off, SCH)], dbuf)

            cur2 = cur
            for j in range(SCH // 16):
                dv = dbuf[pl.ds(16 * j, 16)]
                sv = sbuf[pl.ds(16 * j, 16)]
                msk = (dv >= lo) & (dv < lo + RNG)
                mi = msk.astype(_i32)
                pos = cur2 + plsc.cumsum(mi) - 1
                plsc.store_scatter(fdl, [pos], dv - lo, mask=msk)
                plsc.store_scatter(fsrc, [pos], sv, mask=msk)
                cur2 = cur2 + jnp.sum(mi)
            return cur2

        lax.fori_loop(0, SEGE // SCH, schunk, 0)

        def rmw(g, rows, lbase):
            base = g * 16

            def edge(e):
                dl = lax.gather(
                    fdl[pl.ds(lbase + base, 16)],
                    jnp.full((16, 1), e, _i32),
                    lax.GatherDimensionNumbers(offset_dims=(),
                                               collapsed_slice_dims=(0,),
                                               start_index_map=(0,)),
                    (1,),
                    mode=lax.GatherScatterMode.PROMISE_IN_BOUNDS)
                for f in range(4):
                    col = _iota16() + 16 * f
                    av = plsc.load_gather(acc, [dl, col])
                    mv = rows[base + e, pl.ds(16 * f, 16)]
                    plsc.store_scatter(acc, [dl, col], jnp.minimum(av, mv))

            for e in range(16):
                edge(e)

        def bpair(i, _2):
            b0 = (2 * i) * GB
            d0 = pltpu.async_copy(G_h.at[fsrc.at[pl.ds(b0, GB)]], rows0, sem0)
            b1 = b0 + GB
            d1 = pltpu.async_copy(G_h.at[fsrc.at[pl.ds(b1, GB)]], rows1, sem1)
            d0.wait()

            def g0(g, _3):
                rmw(g, rows0, b0)
                return 0

            lax.fori_loop(0, GB // 16, g0, 0)
            d1.wait()

            def g1(g, _3):
                rmw(g, rows1, b1)
                return 0

            lax.fori_loop(0, GB // 16, g1, 0)
            return 0

        lax.fori_loop(0, CAP // GB // 2, bpair, 0)
        return 0

    lax.fori_loop(0, NSEG, seg, 0)
    pltpu.sync_copy(acc.at[pl.ds(0, RNG)], M_h.at[pl.ds(lo, RNG)])


def _emin_body(*refs):
    (g0, g1, g2, g3, s0, s1, s2, s3, d0, d1, d2, d3,
     m0, m1, m2, m3,
     acc, sbuf, dbuf, fsrc, fdl, rows0, rows1, sem0, sem1) = refs
    c = lax.axis_index("c")
    s = lax.axis_index("s")
    w = c * NSUB + s
    Gs = (g0, g1, g2, g3)
    Ss = (s0, s1, s2, s3)
    Ds = (d0, d1, d2, d3)
    Ms = (m0, m1, m2, m3)
    for t in range(T):
        @pl.when(w // 8 == t)
        def _(t=t):
            _emin_tower(Gs[t], Ss[t], Ds[t], Ms[t], w, t,
                        acc, sbuf, dbuf, fsrc, fdl, rows0, rows1, sem0, sem1)


def _sc_emin(Gp, srcs, dsts):
    f = pl.kernel(
        _emin_body,
        out_type=[jax.ShapeDtypeStruct((NP, H), _f32) for _ in range(T)],
        mesh=_mesh(),
        compiler_params=pltpu.CompilerParams(use_tc_tiling_on_sc=False, needs_layout_passes=False),
        scratch_types=[
            pltpu.VMEM((ACC_R, H), _f32),
            pltpu.VMEM((SCH,), _i32), pltpu.VMEM((SCH,), _i32),
            pltpu.VMEM((CAPB,), _i32), pltpu.VMEM((CAPB,), _i32),
            pltpu.VMEM((GB, H), _f32), pltpu.VMEM((GB, H), _f32),
            pltpu.SemaphoreType.DMA, pltpu.SemaphoreType.DMA,
        ],
    )
    return f(Gp[0], Gp[1], Gp[2], Gp[3],
             srcs[0], srcs[1], srcs[2], srcs[3],
             dsts[0], dsts[1], dsts[2], dsts[3])


def _cls_kernel(z_ref, w1_ref, b1_ref, w2_ref, b2_ref, w3_ref, b3_ref, o_ref):
    z = z_ref[...]
    z = jax.nn.relu(jnp.dot(z, w1_ref[...], preferred_element_type=_f32) + b1_ref[...])
    z = jax.nn.relu(jnp.dot(z, w2_ref[...], preferred_element_type=_f32) + b2_ref[...])
    o_ref[...] = jnp.dot(z, w3_ref[...], preferred_element_type=_f32) + b3_ref[...]


def _classifier(z, w1, b1, w2, b2, w3, b3):
    return pl.pallas_call(
        _cls_kernel,
        out_shape=jax.ShapeDtypeStruct((B, OUT_F), _f32),
    )(z, w1, b1, w2, b2, w3, b3)


def kernel(x0, x1, x2, x3, edge_index0, edge_index1, edge_index2, edge_index3, graph_ids, m0_conv1_W, m0_conv1_b, m0_conv2_W, m0_conv2_b, m0_conv3_W, m0_conv3_b, m0_ec1_tW, m0_ec1_tb, m0_ec1_pW, m0_ec1_pb, m0_ec2_tW, m0_ec2_tb, m0_ec2_pW, m0_ec2_pb, m1_conv1_W, m1_conv1_b, m1_conv2_W, m1_conv2_b, m1_conv3_W, m1_conv3_b, m1_ec1_tW, m1_ec1_tb, m1_ec1_pW, m1_ec1_pb, m1_ec2_tW, m1_ec2_tb, m1_ec2_pW, m1_ec2_pb, m2_conv1_W, m2_conv1_b, m2_conv2_W, m2_conv2_b, m2_conv3_W, m2_conv3_b, m2_ec1_tW, m2_ec1_tb, m2_ec1_pW, m2_ec1_pb, m2_ec2_tW, m2_ec2_tb, m2_ec2_pW, m2_ec2_pb, m3_conv1_W, m3_conv1_b, m3_conv2_W, m3_conv2_b, m3_conv3_W, m3_conv3_b, m3_ec1_tW, m3_ec1_tb, m3_ec1_pW, m3_ec1_pb, m3_ec2_tW, m3_ec2_tb, m3_ec2_pW, m3_ec2_pb, cls_W1, cls_b1, cls_W2, cls_b2, cls_W3, cls_b3):
    fl = dict(locals())
    xs = [x0, x1, x2, x3]
    eis = [edge_index0, edge_index1, edge_index2, edge_index3]
    srcs = [ei[0] for ei in eis]
    dsts = [ei[1] for ei in eis]

    zeros16 = jnp.zeros((NP, 16), _f32)
    zeros64 = jnp.zeros((NP, H), _f32)
    ones_h = jnp.ones((CH, 16), _f32)
    padrows = jnp.zeros((NP - N, H), _f32)
    pad = lambda a: jnp.concatenate([a, padrows], axis=0)

    deg16 = _sc_deg(dsts, zeros16, ones_h)
    degs = [d[:N, 0] for d in deg16]
    norms = [jnp.power(jnp.clip(d, 1.0), -0.5)[:, None] for d in degs]
    nsq64 = [pad(jnp.broadcast_to(nm * nm, (N, H))) for nm in norms]

    def cheb_stage(hs, Wn, bn):
        Bp, A, C = [], [], []
        for t in range(T):
            W = fl[f"m{t}_{Wn}"]
            b = fl[f"m{t}_{bn}"]
            F = hs[t].shape[1]
            W0, W1, W2 = W[:F], W[F:2 * F], W[2 * F:]
            C.append(hs[t] @ (W0 - W2) + b)
            A.append(pad(norms[t] * (hs[t] @ W1)))
            Bp.append(pad(2.0 * (norms[t] * (hs[t] @ W2))))
        S = _sc_cheb(Bp, A, nsq64, srcs, dsts, zeros64)
        return [jax.nn.relu(C[t] + norms[t] * S[t][:N]) for t in range(T)]

    def emin_stage(hs, tWn, tbn, pWn, pbn):
        Gp, Q = [], []
        pad = jnp.full((16, H), BIG, _f32)
        for t in range(T):
            tW = fl[f"m{t}_{tWn}"]
            pW = fl[f"m{t}_{pWn}"]
            cst = fl[f"m{t}_{tbn}"] + fl[f"m{t}_{pbn}"]
            Gp.append(jnp.concatenate([hs[t] @ tW, pad], axis=0))
            Q.append(hs[t] @ (tW + pW) + cst)
        M = _sc_emin(Gp, srcs, dsts)
        return [jax.nn.relu(jnp.where(degs[t][:, None] > 0, Q[t] - M[t][:N], 0.0))
                for t in range(T)]

    h = cheb_stage(xs, "conv1_W", "conv1_b")
    h = emin_stage(h, "ec1_tW", "ec1_tb", "ec1_pW", "ec1_pb")
    h = cheb_stage(h, "conv2_W", "conv2_b")
    h = emin_stage(h, "ec2_tW", "ec2_tb", "ec2_pW", "ec2_pb")
    h = cheb_stage(h, "conv3_W", "conv3_b")

    gid = graph_ids
    cnt = jnp.clip(jax.ops.segment_sum(jnp.ones((N,), _f32), gid, num_segments=B), 1.0)[:, None]
    onehot = (gid[None, :] == jnp.arange(B, dtype=_i32)[:, None]).astype(_f32)
    reps = [(onehot @ h[t]) / cnt for t in range(T)]
    z = jnp.concatenate(reps, axis=1)
    return _classifier(z, cls_W1, cls_b1, cls_W2, cls_b2, cls_W3, cls_b3)
